# baseline (speedup 1.0000x reference)
import numpy as np
import jax
from jax import lax
import jax.numpy as jnp
from jax.experimental import pallas as pl
from jax.experimental.pallas import tpu as pltpu

_MAX_WAVELENGTH = 10000.0


def _sine_pe_np(seq_len: int, dim: int) -> np.ndarray:
    position = np.arange(seq_len, dtype=np.float64)
    min_freq = 1.0 / _MAX_WAVELENGTH
    timescales = np.power(
        min_freq,
        (2 * (np.arange(dim) // 2)).astype(np.float64) / float(dim),
    )
    angles = position[:, None] * timescales[None, :]
    cos_mask = (np.arange(dim) % 2).astype(np.float64)
    pe = np.sin(angles) * (1.0 - cos_mask) + np.cos(angles) * cos_mask
    return pe.astype(np.float32)


_NBUF = 3


def _make_body(bb, nbuf, nsplit):
    sb = bb // nsplit

    def body(x_hbm, pe_ref, o_hbm, bin_ref, bout_ref, sin, sout):
        i = pl.program_id(0)
        n = pl.num_programs(0)

        class _Group:
            def __init__(self, copies):
                self.copies = copies

            def start(self):
                for c in self.copies:
                    c.start()

            def wait(self):
                for c in self.copies:
                    c.wait()

        def in_copy(j, slot):
            return _Group([
                pltpu.make_async_copy(
                    x_hbm.at[pl.ds(j * bb + s * sb, sb)],
                    bin_ref.at[slot, pl.ds(s * sb, sb)],
                    sin.at[slot, s])
                for s in range(nsplit)
            ])

        def out_copy(j, slot):
            return _Group([
                pltpu.make_async_copy(
                    bout_ref.at[slot, pl.ds(s * sb, sb)],
                    o_hbm.at[pl.ds(j * bb + s * sb, sb)],
                    sout.at[slot, s])
                for s in range(nsplit)
            ])

        slot = lax.rem(i, nbuf)

        @pl.when(i == 0)
        def _():
            for s in range(nbuf - 1):
                in_copy(s, s).start()

        nxt = i + nbuf - 1

        @pl.when(nxt < n)
        def _():
            in_copy(nxt, lax.rem(nxt, nbuf)).start()

        in_copy(i, slot).wait()

        @pl.when(i >= nbuf)
        def _():
            out_copy(i - nbuf, slot).wait()

        bout_ref[slot] = bin_ref[slot] + pe_ref[...]
        out_copy(i, slot).start()

        @pl.when(i == n - 1)
        def _():
            for k in range(nbuf):
                j = n - nbuf + k
                out_copy(j, lax.rem(j, nbuf)).wait()

    return body


def kernel(inputs, table):
    batch, seq_len, dim = inputs.shape
    pe = jnp.asarray(_sine_pe_np(seq_len, dim)[None])

    bb = 256
    nbuf = _NBUF
    nsplit = 4
    grid = (batch // bb,)
    out = pl.pallas_call(
        _make_body(bb, nbuf, nsplit),
        grid=grid,
        in_specs=[
            pl.BlockSpec(memory_space=pl.ANY),
            pl.BlockSpec((1, seq_len, dim), lambda i: (0, 0, 0)),
        ],
        out_specs=pl.BlockSpec(memory_space=pl.ANY),
        out_shape=jax.ShapeDtypeStruct((batch, seq_len, dim), jnp.float32),
        scratch_shapes=[
            pltpu.VMEM((nbuf, bb, seq_len, dim), jnp.float32),
            pltpu.VMEM((nbuf, bb, seq_len, dim), jnp.float32),
            pltpu.SemaphoreType.DMA((nbuf, nsplit)),
            pltpu.SemaphoreType.DMA((nbuf, nsplit)),
        ],
    )(inputs, pe)
    return out
